# Initial kernel scaffold; baseline (speedup 1.0000x reference)
import jax
import jax.numpy as jnp
from jax.experimental import pallas as pl


def kernel(node_features, edge_index, atom_weights, batch, atom_W, atom_b, edge_W, edge_b, mlp_W, mlp_b, nn_W, nn_b, bn_gamma, bn_beta, out_W, out_b):
    raise NotImplementedError("write your pallas kernel here")


# speedup vs baseline: 1.1474x; 1.1474x over previous
import functools

import jax
import jax.numpy as jnp
from jax.experimental import pallas as pl
from jax.experimental.pallas import tpu as pltpu

N = 50000
E = 800000
G = 256
D = 256
BN = 2000
NB = N // BN
EPS = 1e-5
LAYERS = 4


def _x0_body(nf_ref, w_ref, b_ref, x_ref):
    x_ref[...] = jnp.dot(nf_ref[...], w_ref[...],
                         preferred_element_type=jnp.float32) + b_ref[...]


def _x0(nf_pad, atom_Wp, atom_b):
    return pl.pallas_call(
        _x0_body,
        grid=(NB,),
        in_specs=[
            pl.BlockSpec((BN, 128), lambda i: (i, 0)),
            pl.BlockSpec((128, D), lambda i: (0, 0)),
            pl.BlockSpec((1, D), lambda i: (0, 0)),
        ],
        out_specs=pl.BlockSpec((BN, D), lambda i: (i, 0)),
        out_shape=jax.ShapeDtypeStruct((N, D), jnp.float32),
    )(nf_pad, atom_Wp, atom_b)


def _vc_body(ew_ref, eb_ref, mb_ref, w3_ref, vs_ref, cs_ref):
    w3 = w3_ref[...]
    v = ew_ref[...]
    c = eb_ref[...]
    vrows, crows = [], []
    for _ in range(LAYERS):
        v = jnp.dot(v, w3, preferred_element_type=jnp.float32)
        c = jnp.dot(c, w3, preferred_element_type=jnp.float32) + mb_ref[...]
        vrows.append(v)
        crows.append(c)
    vs_ref[...] = jnp.concatenate(vrows, axis=0)
    cs_ref[...] = jnp.concatenate(crows, axis=0)


def _vc(edge_W, edge_b, mlp_b, W3):
    return pl.pallas_call(
        _vc_body,
        in_specs=[pl.BlockSpec((1, D), lambda: (0, 0)),
                  pl.BlockSpec((1, D), lambda: (0, 0)),
                  pl.BlockSpec((1, D), lambda: (0, 0)),
                  pl.BlockSpec((D, D), lambda: (0, 0))],
        out_specs=[pl.BlockSpec((LAYERS, D), lambda: (0, 0)),
                   pl.BlockSpec((LAYERS, D), lambda: (0, 0))],
        out_shape=[jax.ShapeDtypeStruct((LAYERS, D), jnp.float32),
                   jax.ShapeDtypeStruct((LAYERS, D), jnp.float32)],
    )(edge_W, edge_b.reshape(1, D), mlp_b.reshape(1, D), W3)


def _norm_x(hx, stats, gamma, beta):
    mean = stats[0:1] * (1.0 / N)
    var = stats[1:2] * (1.0 / N) - mean * mean
    inv = jax.lax.rsqrt(var + EPS)
    return jax.nn.relu((hx - mean) * inv * gamma + beta)


def _prep_body(first, hx_ref, stats_ref, g_ref, b_ref, p_ref, q_ref,
               w1_ref, w2_ref, w3_ref, c_ref,
               pn_ref, qn_ref, u_ref, v_ref):
    if first:
        x = hx_ref[...]
        pn = jnp.dot(x, w1_ref[...], preferred_element_type=jnp.float32)
        qn = jnp.dot(x, w2_ref[...], preferred_element_type=jnp.float32)
    else:
        x = _norm_x(hx_ref[...], stats_ref[...], g_ref[...], b_ref[...])
        w3 = w3_ref[...]
        pn = (jnp.dot(x, w1_ref[...], preferred_element_type=jnp.float32)
              + jnp.dot(p_ref[...], w3, preferred_element_type=jnp.float32))
        qn = (jnp.dot(x, w2_ref[...], preferred_element_type=jnp.float32)
              + jnp.dot(q_ref[...], w3, preferred_element_type=jnp.float32))
    pn_ref[...] = pn
    qn_ref[...] = qn
    u_ref[...] = x + pn
    v_ref[...] = qn + c_ref[...]


def _prep(first, hx, stats, gamma, beta, P, Q, W1, W2, W3, c_l):
    return pl.pallas_call(
        functools.partial(_prep_body, first),
        grid=(NB,),
        in_specs=[
            pl.BlockSpec((BN, D), lambda i: (i, 0)),
            pl.BlockSpec((8, D), lambda i: (0, 0)),
            pl.BlockSpec((1, D), lambda i: (0, 0)),
            pl.BlockSpec((1, D), lambda i: (0, 0)),
            pl.BlockSpec((BN, D), lambda i: (i, 0)),
            pl.BlockSpec((BN, D), lambda i: (i, 0)),
            pl.BlockSpec((D, D), lambda i: (0, 0)),
            pl.BlockSpec((D, D), lambda i: (0, 0)),
            pl.BlockSpec((D, D), lambda i: (0, 0)),
            pl.BlockSpec((1, D), lambda i: (0, 0)),
        ],
        out_specs=[pl.BlockSpec((BN, D), lambda i: (i, 0))] * 4,
        out_shape=[jax.ShapeDtypeStruct((N, D), jnp.float32)] * 4,
    )(hx, stats, gamma, beta, P, Q, W1, W2, W3, c_l)


def _dense2_body(first, agg_ref, hx_ref, stats_ref, g_ref, b_ref,
                 w_ref, bias_ref, h_ref, so_ref):
    i = pl.program_id(0)
    if first:
        x = hx_ref[...]
    else:
        x = _norm_x(hx_ref[...], stats_ref[...], g_ref[...], b_ref[...])
    h = jnp.dot(agg_ref[...] + x, w_ref[...],
                preferred_element_type=jnp.float32) + bias_ref[...]
    h_ref[...] = h
    s = jnp.sum(h, axis=0, keepdims=True)
    s2 = jnp.sum(h * h, axis=0, keepdims=True)
    blk = jnp.concatenate([s, s2, jnp.zeros((6, D), jnp.float32)], axis=0)

    @pl.when(i == 0)
    def _():
        so_ref[...] = blk

    @pl.when(i > 0)
    def _():
        so_ref[...] += blk


def _dense2(first, agg, hx, stats, gamma, beta, nn_W, nn_b):
    return pl.pallas_call(
        functools.partial(_dense2_body, first),
        grid=(NB,),
        in_specs=[
            pl.BlockSpec((BN, D), lambda i: (i, 0)),
            pl.BlockSpec((BN, D), lambda i: (i, 0)),
            pl.BlockSpec((8, D), lambda i: (0, 0)),
            pl.BlockSpec((1, D), lambda i: (0, 0)),
            pl.BlockSpec((1, D), lambda i: (0, 0)),
            pl.BlockSpec((D, D), lambda i: (0, 0)),
            pl.BlockSpec((1, D), lambda i: (0, 0)),
        ],
        out_specs=[pl.BlockSpec((BN, D), lambda i: (i, 0)),
                   pl.BlockSpec((8, D), lambda i: (0, 0))],
        out_shape=[jax.ShapeDtypeStruct((N, D), jnp.float32),
                   jax.ShapeDtypeStruct((8, D), jnp.float32)],
    )(agg, hx, stats, gamma, beta, nn_W, nn_b)


def _pool_body(hx_ref, stats_ref, g_ref, b_ref, batch_ref, ow_ref, ob_ref,
               out_ref, acc_ref):
    i = pl.program_id(0)
    x = _norm_x(hx_ref[...], stats_ref[...], g_ref[...], b_ref[...])
    seg = batch_ref[0, 0]
    onehot = (seg[:, None] ==
              jax.lax.broadcasted_iota(jnp.int32, (BN, G), 1)
              ).astype(jnp.float32)
    part = jax.lax.dot_general(onehot, x, (((0,), (0,)), ((), ())),
                               preferred_element_type=jnp.float32)

    @pl.when(i == 0)
    def _():
        acc_ref[...] = part

    @pl.when(i > 0)
    def _():
        acc_ref[...] += part

    @pl.when(i == NB - 1)
    def _():
        out_ref[...] = jnp.dot(acc_ref[...], ow_ref[...],
                               preferred_element_type=jnp.float32) + ob_ref[...]


def _pool(hx, stats, gamma, beta, batch3, out_W, out_b):
    return pl.pallas_call(
        _pool_body,
        grid=(NB,),
        in_specs=[
            pl.BlockSpec((BN, D), lambda i: (i, 0)),
            pl.BlockSpec((8, D), lambda i: (0, 0)),
            pl.BlockSpec((1, D), lambda i: (0, 0)),
            pl.BlockSpec((1, D), lambda i: (0, 0)),
            pl.BlockSpec((1, 1, BN), lambda i: (i, 0, 0)),
            pl.BlockSpec((D, 6), lambda i: (0, 0)),
            pl.BlockSpec((1, 6), lambda i: (0, 0)),
        ],
        out_specs=pl.BlockSpec((G, 6), lambda i: (0, 0)),
        out_shape=jax.ShapeDtypeStruct((G, 6), jnp.float32),
        scratch_shapes=[pltpu.VMEM((G, G), jnp.float32)],
    )(hx, stats, gamma, beta, batch3, out_W, out_b)


def _edge_phase(U, V, row, col, aw, v_l):
    msg = jax.nn.relu(U[row] + V[col] + aw[:, None] * v_l[None, :])
    return jax.ops.segment_sum(msg, col, num_segments=N)


def kernel(node_features, edge_index, atom_weights, batch, atom_W, atom_b,
           edge_W, edge_b, mlp_W, mlp_b, nn_W, nn_b, bn_gamma, bn_beta,
           out_W, out_b):
    f32 = jnp.float32
    W1, W2, W3 = mlp_W[:D], mlp_W[D:2 * D], mlp_W[2 * D:]
    row = edge_index[0]
    col = edge_index[1]
    aw = atom_weights[:, 0]

    nf_pad = jnp.pad(node_features, ((0, 0), (0, 128 - node_features.shape[1])))
    atom_Wp = jnp.pad(atom_W, ((0, 128 - atom_W.shape[0]), (0, 0)))
    gamma = bn_gamma.reshape(1, D)
    beta = bn_beta.reshape(1, D)
    batch3 = batch.astype(jnp.int32).reshape(NB, 1, BN)

    vs, cs = _vc(edge_W, edge_b, mlp_b, W3)

    hx = _x0(nf_pad, atom_Wp, atom_b.reshape(1, D))
    stats = jnp.zeros((8, D), f32)
    P = Q = jnp.zeros((N, D), f32)
    for l in range(LAYERS):
        first = (l == 0)
        P, Q, U, V = _prep(first, hx, stats, gamma, beta, P, Q,
                           W1, W2, W3, cs[l].reshape(1, D))
        agg = _edge_phase(U, V, row, col, aw, vs[l])
        hx, stats = _dense2(first, agg, hx, stats, gamma, beta,
                            nn_W, nn_b.reshape(1, D))
    return _pool(hx, stats, gamma, beta, batch3, out_W, out_b.reshape(1, 6))


# speedup vs baseline: 1.4307x; 1.2469x over previous
import functools

import jax
import jax.numpy as jnp
from jax import lax
from jax.experimental import pallas as pl
from jax.experimental.pallas import tpu as pltpu
from jax.experimental.pallas import tpu_sc as plsc

N = 50000
E = 800000
G = 256
D = 256
BN = 2000
NB = N // BN
EPS = 1e-5
LAYERS = 4

NC = 2
NS = 16
NW = NC * NS
OWN = 3136
KB = 16
CAP = 25216
EPW = E // NW
NG = EPW // 16 + 1
EPAD = 800256
EB = 128
CT = 224
CTA = 232
SUBS = 14
AGGP = NS * OWN


def _x0_body(nf_ref, w_ref, b_ref, x_ref):
    x_ref[...] = jnp.dot(nf_ref[...], w_ref[...],
                         preferred_element_type=jnp.float32) + b_ref[...]


def _x0(nf_pad, atom_Wp, atom_b):
    return pl.pallas_call(
        _x0_body,
        grid=(NB,),
        in_specs=[
            pl.BlockSpec((BN, 128), lambda i: (i, 0)),
            pl.BlockSpec((128, D), lambda i: (0, 0)),
            pl.BlockSpec((1, D), lambda i: (0, 0)),
        ],
        out_specs=pl.BlockSpec((BN, D), lambda i: (i, 0)),
        out_shape=jax.ShapeDtypeStruct((N, D), jnp.float32),
    )(nf_pad, atom_Wp, atom_b)


def _vc_body(ew_ref, eb_ref, mb_ref, w3_ref, vs_ref, cs_ref):
    w3 = w3_ref[...]
    v = ew_ref[...]
    c = eb_ref[...]
    vrows, crows = [], []
    for _ in range(LAYERS):
        v = jnp.dot(v, w3, preferred_element_type=jnp.float32)
        c = jnp.dot(c, w3, preferred_element_type=jnp.float32) + mb_ref[...]
        vrows.append(v)
        crows.append(c)
    vs_ref[...] = jnp.concatenate(vrows, axis=0)
    cs_ref[...] = jnp.concatenate(crows, axis=0)


def _vc(edge_W, edge_b, mlp_b, W3):
    return pl.pallas_call(
        _vc_body,
        in_specs=[pl.BlockSpec((1, D), lambda: (0, 0)),
                  pl.BlockSpec((1, D), lambda: (0, 0)),
                  pl.BlockSpec((1, D), lambda: (0, 0)),
                  pl.BlockSpec((D, D), lambda: (0, 0))],
        out_specs=[pl.BlockSpec((LAYERS, D), lambda: (0, 0)),
                   pl.BlockSpec((LAYERS, D), lambda: (0, 0))],
        out_shape=[jax.ShapeDtypeStruct((LAYERS, D), jnp.float32),
                   jax.ShapeDtypeStruct((LAYERS, D), jnp.float32)],
    )(edge_W, edge_b.reshape(1, D), mlp_b.reshape(1, D), W3)


def _norm_x(hx, stats, gamma, beta):
    mean = stats[0:1] * (1.0 / N)
    var = stats[1:2] * (1.0 / N) - mean * mean
    inv = jax.lax.rsqrt(var + EPS)
    return jax.nn.relu((hx - mean) * inv * gamma + beta)


def _prep_body(first, hx_ref, stats_ref, g_ref, b_ref, p_ref, q_ref,
               w1_ref, w2_ref, w3_ref, c_ref,
               pn_ref, qn_ref, u_ref, v_ref):
    if first:
        x = hx_ref[...]
        pn = jnp.dot(x, w1_ref[...], preferred_element_type=jnp.float32)
        qn = jnp.dot(x, w2_ref[...], preferred_element_type=jnp.float32)
    else:
        x = _norm_x(hx_ref[...], stats_ref[...], g_ref[...], b_ref[...])
        w3 = w3_ref[...]
        pn = (jnp.dot(x, w1_ref[...], preferred_element_type=jnp.float32)
              + jnp.dot(p_ref[...], w3, preferred_element_type=jnp.float32))
        qn = (jnp.dot(x, w2_ref[...], preferred_element_type=jnp.float32)
              + jnp.dot(q_ref[...], w3, preferred_element_type=jnp.float32))
    pn_ref[...] = pn
    qn_ref[...] = qn
    u_ref[...] = x + pn
    v_ref[...] = qn + c_ref[...]


def _prep(first, hx, stats, gamma, beta, P, Q, W1, W2, W3, c_l):
    return pl.pallas_call(
        functools.partial(_prep_body, first),
        grid=(NB,),
        in_specs=[
            pl.BlockSpec((BN, D), lambda i: (i, 0)),
            pl.BlockSpec((8, D), lambda i: (0, 0)),
            pl.BlockSpec((1, D), lambda i: (0, 0)),
            pl.BlockSpec((1, D), lambda i: (0, 0)),
            pl.BlockSpec((BN, D), lambda i: (i, 0)),
            pl.BlockSpec((BN, D), lambda i: (i, 0)),
            pl.BlockSpec((D, D), lambda i: (0, 0)),
            pl.BlockSpec((D, D), lambda i: (0, 0)),
            pl.BlockSpec((D, D), lambda i: (0, 0)),
            pl.BlockSpec((1, D), lambda i: (0, 0)),
        ],
        out_specs=[pl.BlockSpec((BN, D), lambda i: (i, 0))] * 4,
        out_shape=[jax.ShapeDtypeStruct((N, D), jnp.float32)] * 4,
    )(hx, stats, gamma, beta, P, Q, W1, W2, W3, c_l)


def _dense2_body(first, agg0_ref, agg1_ref, hx_ref, stats_ref, g_ref, b_ref,
                 w_ref, bias_ref, h_ref, so_ref):
    i = pl.program_id(0)
    if first:
        x = hx_ref[...]
    else:
        x = _norm_x(hx_ref[...], stats_ref[...], g_ref[...], b_ref[...])
    h = jnp.dot(agg0_ref[...] + agg1_ref[...] + x, w_ref[...],
                preferred_element_type=jnp.float32) + bias_ref[...]
    h_ref[...] = h
    s = jnp.sum(h, axis=0, keepdims=True)
    s2 = jnp.sum(h * h, axis=0, keepdims=True)
    blk = jnp.concatenate([s, s2, jnp.zeros((6, D), jnp.float32)], axis=0)

    @pl.when(i == 0)
    def _():
        so_ref[...] = blk

    @pl.when(i > 0)
    def _():
        so_ref[...] += blk


def _dense2(first, agg0, agg1, hx, stats, gamma, beta, nn_W, nn_b):
    return pl.pallas_call(
        functools.partial(_dense2_body, first),
        grid=(NB,),
        in_specs=[
            pl.BlockSpec((BN, D), lambda i: (i, 0)),
            pl.BlockSpec((BN, D), lambda i: (i, 0)),
            pl.BlockSpec((BN, D), lambda i: (i, 0)),
            pl.BlockSpec((8, D), lambda i: (0, 0)),
            pl.BlockSpec((1, D), lambda i: (0, 0)),
            pl.BlockSpec((1, D), lambda i: (0, 0)),
            pl.BlockSpec((D, D), lambda i: (0, 0)),
            pl.BlockSpec((1, D), lambda i: (0, 0)),
        ],
        out_specs=[pl.BlockSpec((BN, D), lambda i: (i, 0)),
                   pl.BlockSpec((8, D), lambda i: (0, 0))],
        out_shape=[jax.ShapeDtypeStruct((N, D), jnp.float32),
                   jax.ShapeDtypeStruct((8, D), jnp.float32)],
    )(agg0, agg1, hx, stats, gamma, beta, nn_W, nn_b)


def _pool_body(hx_ref, stats_ref, g_ref, b_ref, batch_ref, ow_ref, ob_ref,
               out_ref, acc_ref):
    i = pl.program_id(0)
    x = _norm_x(hx_ref[...], stats_ref[...], g_ref[...], b_ref[...])
    seg = batch_ref[0, 0]
    onehot = (seg[:, None] ==
              jax.lax.broadcasted_iota(jnp.int32, (BN, G), 1)
              ).astype(jnp.float32)
    part = jax.lax.dot_general(onehot, x, (((0,), (0,)), ((), ())),
                               preferred_element_type=jnp.float32)

    @pl.when(i == 0)
    def _():
        acc_ref[...] = part

    @pl.when(i > 0)
    def _():
        acc_ref[...] += part

    @pl.when(i == NB - 1)
    def _():
        out_ref[...] = jnp.dot(acc_ref[...], ow_ref[...],
                               preferred_element_type=jnp.float32) + ob_ref[...]


def _pool(hx, stats, gamma, beta, batch3, out_W, out_b):
    return pl.pallas_call(
        _pool_body,
        grid=(NB,),
        in_specs=[
            pl.BlockSpec((BN, D), lambda i: (i, 0)),
            pl.BlockSpec((8, D), lambda i: (0, 0)),
            pl.BlockSpec((1, D), lambda i: (0, 0)),
            pl.BlockSpec((1, D), lambda i: (0, 0)),
            pl.BlockSpec((1, 1, BN), lambda i: (i, 0, 0)),
            pl.BlockSpec((D, 6), lambda i: (0, 0)),
            pl.BlockSpec((1, 6), lambda i: (0, 0)),
        ],
        out_specs=pl.BlockSpec((G, 6), lambda i: (0, 0)),
        out_shape=jax.ShapeDtypeStruct((G, 6), jnp.float32),
        scratch_shapes=[pltpu.VMEM((G, G), jnp.float32)],
    )(hx, stats, gamma, beta, batch3, out_W, out_b)


_MESH = plsc.VectorSubcoreMesh(core_axis_name="c", subcore_axis_name="s")
_SC_PARAMS = pltpu.CompilerParams(needs_layout_passes=False)


def _bin_body(row_hbm, col_hbm, aw_hbm, br_hbm, bc_hbm, ba_hbm, cnt_hbm,
              st_r, st_c, st_a, buf_r, buf_c, buf_a, cv_v):
    wid = lax.axis_index("c") * NS + lax.axis_index("s")
    ebase = wid * EPW
    iota = lax.broadcasted_iota(jnp.int32, (16,), 0)

    def group(g, carry):
        @pl.when(lax.rem(g, 32) == 0)
        def _():
            soff = pl.multiple_of(ebase + (g // 32) * 512, 8)
            pltpu.sync_copy(row_hbm.at[pl.ds(soff, 512)], st_r)
            pltpu.sync_copy(col_hbm.at[pl.ds(soff, 512)], st_c)
            pltpu.sync_copy(aw_hbm.at[pl.ds(soff, 512)], st_a)

        gl = lax.rem(g, 32) * 16
        rv = st_r[pl.ds(gl, 16)]
        cv = st_c[pl.ds(gl, 16)]
        av = st_a[pl.ds(gl, 16)]
        nvalid = jnp.where(g == NG - 1, EPW - (NG - 1) * 16, 16)
        valid = iota < nvalid
        bkt = cv // OWN
        out = []
        for k in range(KB):
            ck, hk = carry[2 * k], carry[2 * k + 1]
            m = (bkt == k) & valid
            plsc.store_compressed(buf_r.at[pl.ds(k * 160 + ck, 16)], rv, mask=m)
            plsc.store_compressed(buf_c.at[pl.ds(k * 160 + ck, 16)], cv, mask=m)
            plsc.store_compressed(buf_a.at[pl.ds(k * 160 + ck, 16)], av, mask=m)
            pc = plsc.all_reduce_population_count(m)
            if getattr(pc, "ndim", 0):
                pc = jnp.max(pc)
            ck = ck + pc
            do = ck >= EB
            dst = (wid * KB + k) * CAP

            @pl.when(do)
            def _(k=k, hk=hk, dst=dst):
                doff = pl.multiple_of(dst + hk, 8)
                pltpu.sync_copy(buf_r.at[pl.ds(k * 160, EB)],
                                br_hbm.at[pl.ds(doff, EB)])
                pltpu.sync_copy(buf_c.at[pl.ds(k * 160, EB)],
                                bc_hbm.at[pl.ds(doff, EB)])
                pltpu.sync_copy(buf_a.at[pl.ds(k * 160, EB)],
                                ba_hbm.at[pl.ds(doff, EB)])
                buf_r[pl.ds(k * 160, 16)] = buf_r[pl.ds(k * 160 + EB, 16)]
                buf_c[pl.ds(k * 160, 16)] = buf_c[pl.ds(k * 160 + EB, 16)]
                buf_a[pl.ds(k * 160, 16)] = buf_a[pl.ds(k * 160 + EB, 16)]

            out.append(jnp.where(do, ck - EB, ck))
            out.append(jnp.where(do, hk + EB, hk))
        return tuple(out)

    carry = lax.fori_loop(0, NG, group, (jnp.int32(0),) * (2 * KB))

    cvec = jnp.zeros((16,), jnp.int32)
    for k in range(KB):
        ck, hk = carry[2 * k], carry[2 * k + 1]
        doff = pl.multiple_of((wid * KB + k) * CAP + hk, 8)
        pltpu.sync_copy(buf_r.at[pl.ds(k * 160, EB)],
                        br_hbm.at[pl.ds(doff, EB)])
        pltpu.sync_copy(buf_c.at[pl.ds(k * 160, EB)],
                        bc_hbm.at[pl.ds(doff, EB)])
        pltpu.sync_copy(buf_a.at[pl.ds(k * 160, EB)],
                        ba_hbm.at[pl.ds(doff, EB)])
        cvec = jnp.where(iota == k, ck + hk, cvec)
    cv_v[...] = cvec
    pltpu.sync_copy(cv_v, cnt_hbm.at[pl.ds(pl.multiple_of(wid * 16, 8), 16)])


_bin = pl.kernel(
    _bin_body,
    out_type=[jax.ShapeDtypeStruct((NW * KB * CAP,), jnp.int32),
              jax.ShapeDtypeStruct((NW * KB * CAP,), jnp.int32),
              jax.ShapeDtypeStruct((NW * KB * CAP,), jnp.float32),
              jax.ShapeDtypeStruct((NW * 16,), jnp.int32)],
    mesh=_MESH,
    compiler_params=_SC_PARAMS,
    scratch_types=[pltpu.VMEM((512,), jnp.int32),
                   pltpu.VMEM((512,), jnp.int32),
                   pltpu.VMEM((512,), jnp.float32),
                   pltpu.VMEM((KB * 160,), jnp.int32),
                   pltpu.VMEM((KB * 160,), jnp.int32),
                   pltpu.VMEM((KB * 160,), jnp.float32),
                   pltpu.VMEM((16,), jnp.int32)],
)


def _edge_body(u_hbm, v_hbm, vl_hbm, br_hbm, bc_hbm, ba_hbm, cnt_hbm,
               zin_hbm, agg_hbm,
               st_r, st_c, st_a, rbuf, cbuf, abuf, lbuf,
               ubuf, vbuf, vlv, cntv, acc, sem_u, sem_v):
    c_ax = lax.axis_index("c")
    s_ax = lax.axis_index("s")
    iota = lax.broadcasted_iota(jnp.int32, (16,), 0)
    pltpu.sync_copy(vl_hbm, vlv)
    pltpu.sync_copy(cnt_hbm, cntv)
    vl_ts = [vlv[pl.ds(t * 16, 16)] for t in range(16)]
    colv_ts = [iota + t * 16 for t in range(16)]
    tile_base = s_ax * OWN
    ns = []
    for w in range(NS):
        cvec = cntv[pl.ds(pl.multiple_of((c_ax * NS + w) * 16, 8), 16)]
        ns.append(jnp.sum(jnp.where(iota == s_ax, cvec, 0)))

    def fire(ck, sub_base, tail):
        for t in range(EB // 16):
            rv = jnp.clip(rbuf[pl.ds(t * 16, 16)], 0, N - 1)
            rbuf[pl.ds(t * 16, 16)] = rv
            cv = jnp.clip(cbuf[pl.ds(t * 16, 16)], 0, N - 1)
            cbuf[pl.ds(t * 16, 16)] = cv
            lv = cv - sub_base
            if tail:
                lv = jnp.where(iota + t * 16 < ck, lv, CT)
            lbuf[pl.ds(t * 16, 16)] = jnp.clip(lv, 0, CT)
        cpu_ = pltpu.async_copy(u_hbm.at[rbuf.at[pl.ds(0, EB)]], ubuf, sem_u)
        cpv_ = pltpu.async_copy(v_hbm.at[cbuf.at[pl.ds(0, EB)]], vbuf, sem_v)
        cpu_.wait()
        cpv_.wait()

        def edge_j(jv, _):
            jj = jv // 16
            jl = jv - jj * 16
            av = abuf[pl.ds(pl.multiple_of(jj * 16, 8), 16)]
            aj = jnp.sum(jnp.where(iota == jl, av, 0.0))
            lv = lbuf[pl.ds(pl.multiple_of(jj * 16, 8), 16)]
            lj = jnp.sum(jnp.where(iota == jl, lv, 0))
            rowv = jnp.zeros((16,), jnp.int32) + lj
            for t in range(16):
                m = ubuf[jv, pl.ds(t * 16, 16)] + vbuf[jv, pl.ds(t * 16, 16)]
                m = jnp.maximum(m + aj * vl_ts[t], 0.0)
                plsc.addupdate_scatter(acc, [rowv, colv_ts[t]], m)
            return 0

        lax.fori_loop(0, EB, edge_j, 0)

    def sub_body(sub, _):
        pltpu.sync_copy(zin_hbm, acc)
        sub_lo = sub * CT

        def w_body(w, ck):
            cvec = cntv[pl.ds(pl.multiple_of((c_ax * NS + w) * 16, 8), 16)]
            n_w = jnp.sum(jnp.where(iota == s_ax, cvec, 0))
            loff = ((c_ax * NS + w) * KB + s_ax) * CAP

            def stage(st, ck):
                soff = pl.multiple_of(loff + st * 512, 8)
                pltpu.sync_copy(br_hbm.at[pl.ds(soff, 512)], st_r)
                pltpu.sync_copy(bc_hbm.at[pl.ds(soff, 512)], st_c)
                pltpu.sync_copy(ba_hbm.at[pl.ds(soff, 512)], st_a)

                def group(g, ck):
                    gl = g * 16
                    rv = st_r[pl.ds(gl, 16)]
                    cv = st_c[pl.ds(gl, 16)]
                    av = st_a[pl.ds(gl, 16)]
                    glane = st * 512 + gl + iota
                    lrow = cv - tile_base - sub_lo
                    m = (glane < n_w) & (lrow >= 0) & (lrow < CT)
                    plsc.store_compressed(rbuf.at[pl.ds(ck, 16)], rv, mask=m)
                    plsc.store_compressed(cbuf.at[pl.ds(ck, 16)], cv, mask=m)
                    plsc.store_compressed(abuf.at[pl.ds(ck, 16)], av, mask=m)
                    pc = plsc.all_reduce_population_count(m)
                    if getattr(pc, "ndim", 0):
                        pc = jnp.max(pc)
                    ck = ck + pc
                    do = ck >= EB

                    @pl.when(do)
                    def _():
                        fire(EB, tile_base + sub_lo, False)
                        rbuf[pl.ds(0, 16)] = rbuf[pl.ds(EB, 16)]
                        cbuf[pl.ds(0, 16)] = cbuf[pl.ds(EB, 16)]
                        abuf[pl.ds(0, 16)] = abuf[pl.ds(EB, 16)]

                    return jnp.where(do, ck - EB, ck)

                return lax.fori_loop(0, 32, group, ck)

            return lax.fori_loop(0, (n_w + 511) // 512, stage, ck)

        ck = lax.fori_loop(0, NS, w_body, jnp.int32(0))

        @pl.when(ck > 0)
        def _():
            fire(ck, tile_base + sub_lo, True)

        dst = pl.multiple_of(c_ax * AGGP + tile_base + sub_lo, 8)
        pltpu.sync_copy(acc.at[pl.ds(0, CT)], agg_hbm.at[pl.ds(dst, CT)])
        return 0

    lax.fori_loop(0, SUBS, sub_body, 0)


_edge_sc = pl.kernel(
    _edge_body,
    out_type=jax.ShapeDtypeStruct((NC * AGGP, D), jnp.float32),
    mesh=_MESH,
    compiler_params=_SC_PARAMS,
    scratch_types=[pltpu.VMEM((512,), jnp.int32),
                   pltpu.VMEM((512,), jnp.int32),
                   pltpu.VMEM((512,), jnp.float32),
                   pltpu.VMEM((EB + 32,), jnp.int32),
                   pltpu.VMEM((EB + 32,), jnp.int32),
                   pltpu.VMEM((EB + 32,), jnp.float32),
                   pltpu.VMEM((EB,), jnp.int32),
                   pltpu.VMEM((EB, D), jnp.float32),
                   pltpu.VMEM((EB, D), jnp.float32),
                   pltpu.VMEM((D,), jnp.float32),
                   pltpu.VMEM((NW * 16,), jnp.int32),
                   pltpu.VMEM((CTA, D), jnp.float32),
                   pltpu.SemaphoreType.DMA,
                   pltpu.SemaphoreType.DMA],
)


def kernel(node_features, edge_index, atom_weights, batch, atom_W, atom_b,
           edge_W, edge_b, mlp_W, mlp_b, nn_W, nn_b, bn_gamma, bn_beta,
           out_W, out_b):
    f32 = jnp.float32
    W1, W2, W3 = mlp_W[:D], mlp_W[D:2 * D], mlp_W[2 * D:]
    row = edge_index[0]
    col = edge_index[1]
    aw = atom_weights[:, 0]

    nf_pad = jnp.pad(node_features, ((0, 0), (0, 128 - node_features.shape[1])))
    atom_Wp = jnp.pad(atom_W, ((0, 128 - atom_W.shape[0]), (0, 0)))
    gamma = bn_gamma.reshape(1, D)
    beta = bn_beta.reshape(1, D)
    batch3 = batch.astype(jnp.int32).reshape(NB, 1, BN)

    rowp = jnp.pad(row.astype(jnp.int32), (0, EPAD - E))
    colp = jnp.pad(col.astype(jnp.int32), (0, EPAD - E))
    awp = jnp.pad(aw, (0, EPAD - E))
    zin = jnp.zeros((CTA, D), f32)
    br, bc, ba, cnts = _bin(rowp, colp, awp)

    vs, cs = _vc(edge_W, edge_b, mlp_b, W3)

    hx = _x0(nf_pad, atom_Wp, atom_b.reshape(1, D))
    stats = jnp.zeros((8, D), f32)
    P = Q = jnp.zeros((N, D), f32)
    for l in range(LAYERS):
        first = (l == 0)
        P, Q, U, V = _prep(first, hx, stats, gamma, beta, P, Q,
                           W1, W2, W3, cs[l].reshape(1, D))
        agg2 = _edge_sc(U, V, vs[l], br, bc, ba, cnts, zin)
        hx, stats = _dense2(first, agg2[:N], agg2[AGGP:AGGP + N],
                            hx, stats, gamma, beta,
                            nn_W, nn_b.reshape(1, D))
    return _pool(hx, stats, gamma, beta, batch3, out_W, out_b.reshape(1, 6))


# speedup vs baseline: 1.4587x; 1.0195x over previous
import functools

import jax
import jax.numpy as jnp
from jax import lax
from jax.experimental import pallas as pl
from jax.experimental.pallas import tpu as pltpu
from jax.experimental.pallas import tpu_sc as plsc

N = 50000
E = 800000
G = 256
D = 256
BN = 2000
NB = N // BN
EPS = 1e-5
LAYERS = 4

NC = 2
NS = 16
NW = NC * NS
OWN = 3136
KB = 16
CAP = 25216
EPW = E // NW
NG = EPW // 16 + 1
EPAD = 800256
EB = 128
CT = 224
CTA = 232
SUBS = 14
AGGP = NS * OWN


def _x0_body(nf_ref, w_ref, b_ref, x_ref):
    x_ref[...] = jnp.dot(nf_ref[...], w_ref[...],
                         preferred_element_type=jnp.float32) + b_ref[...]


def _x0(nf_pad, atom_Wp, atom_b):
    return pl.pallas_call(
        _x0_body,
        grid=(NB,),
        in_specs=[
            pl.BlockSpec((BN, 128), lambda i: (i, 0)),
            pl.BlockSpec((128, D), lambda i: (0, 0)),
            pl.BlockSpec((1, D), lambda i: (0, 0)),
        ],
        out_specs=pl.BlockSpec((BN, D), lambda i: (i, 0)),
        out_shape=jax.ShapeDtypeStruct((N, D), jnp.float32),
    )(nf_pad, atom_Wp, atom_b)


def _vc_body(ew_ref, eb_ref, mb_ref, w3_ref, vs_ref, cs_ref):
    w3 = w3_ref[...]
    v = ew_ref[...]
    c = eb_ref[...]
    vrows, crows = [], []
    for _ in range(LAYERS):
        v = jnp.dot(v, w3, preferred_element_type=jnp.float32)
        c = jnp.dot(c, w3, preferred_element_type=jnp.float32) + mb_ref[...]
        vrows.append(v)
        crows.append(c)
    vs_ref[...] = jnp.concatenate(vrows, axis=0)
    cs_ref[...] = jnp.concatenate(crows, axis=0)


def _vc(edge_W, edge_b, mlp_b, W3):
    return pl.pallas_call(
        _vc_body,
        in_specs=[pl.BlockSpec((1, D), lambda: (0, 0)),
                  pl.BlockSpec((1, D), lambda: (0, 0)),
                  pl.BlockSpec((1, D), lambda: (0, 0)),
                  pl.BlockSpec((D, D), lambda: (0, 0))],
        out_specs=[pl.BlockSpec((LAYERS, D), lambda: (0, 0)),
                   pl.BlockSpec((LAYERS, D), lambda: (0, 0))],
        out_shape=[jax.ShapeDtypeStruct((LAYERS, D), jnp.float32),
                   jax.ShapeDtypeStruct((LAYERS, D), jnp.float32)],
    )(edge_W, edge_b.reshape(1, D), mlp_b.reshape(1, D), W3)


def _norm_x(hx, stats, gamma, beta):
    mean = stats[0:1] * (1.0 / N)
    var = stats[1:2] * (1.0 / N) - mean * mean
    inv = jax.lax.rsqrt(var + EPS)
    return jax.nn.relu((hx - mean) * inv * gamma + beta)


def _prep_body(first, hx_ref, stats_ref, g_ref, b_ref, p_ref, q_ref,
               w1_ref, w2_ref, w3_ref, c_ref,
               pn_ref, qn_ref, u_ref, v_ref):
    if first:
        x = hx_ref[...]
        pn = jnp.dot(x, w1_ref[...], preferred_element_type=jnp.float32)
        qn = jnp.dot(x, w2_ref[...], preferred_element_type=jnp.float32)
    else:
        x = _norm_x(hx_ref[...], stats_ref[...], g_ref[...], b_ref[...])
        w3 = w3_ref[...]
        pn = (jnp.dot(x, w1_ref[...], preferred_element_type=jnp.float32)
              + jnp.dot(p_ref[...], w3, preferred_element_type=jnp.float32))
        qn = (jnp.dot(x, w2_ref[...], preferred_element_type=jnp.float32)
              + jnp.dot(q_ref[...], w3, preferred_element_type=jnp.float32))
    pn_ref[...] = pn
    qn_ref[...] = qn
    u_ref[...] = x + pn
    v_ref[...] = qn + c_ref[...]


def _prep(first, hx, stats, gamma, beta, P, Q, W1, W2, W3, c_l):
    return pl.pallas_call(
        functools.partial(_prep_body, first),
        grid=(NB,),
        in_specs=[
            pl.BlockSpec((BN, D), lambda i: (i, 0)),
            pl.BlockSpec((8, D), lambda i: (0, 0)),
            pl.BlockSpec((1, D), lambda i: (0, 0)),
            pl.BlockSpec((1, D), lambda i: (0, 0)),
            pl.BlockSpec((BN, D), lambda i: (i, 0)),
            pl.BlockSpec((BN, D), lambda i: (i, 0)),
            pl.BlockSpec((D, D), lambda i: (0, 0)),
            pl.BlockSpec((D, D), lambda i: (0, 0)),
            pl.BlockSpec((D, D), lambda i: (0, 0)),
            pl.BlockSpec((1, D), lambda i: (0, 0)),
        ],
        out_specs=[pl.BlockSpec((BN, D), lambda i: (i, 0))] * 4,
        out_shape=[jax.ShapeDtypeStruct((N, D), jnp.float32)] * 4,
    )(hx, stats, gamma, beta, P, Q, W1, W2, W3, c_l)


def _dense2_body(first, agg0_ref, agg1_ref, hx_ref, stats_ref, g_ref, b_ref,
                 w_ref, bias_ref, h_ref, so_ref):
    i = pl.program_id(0)
    if first:
        x = hx_ref[...]
    else:
        x = _norm_x(hx_ref[...], stats_ref[...], g_ref[...], b_ref[...])
    h = jnp.dot(agg0_ref[...] + agg1_ref[...] + x, w_ref[...],
                preferred_element_type=jnp.float32) + bias_ref[...]
    h_ref[...] = h
    s = jnp.sum(h, axis=0, keepdims=True)
    s2 = jnp.sum(h * h, axis=0, keepdims=True)
    blk = jnp.concatenate([s, s2, jnp.zeros((6, D), jnp.float32)], axis=0)

    @pl.when(i == 0)
    def _():
        so_ref[...] = blk

    @pl.when(i > 0)
    def _():
        so_ref[...] += blk


def _dense2(first, agg0, agg1, hx, stats, gamma, beta, nn_W, nn_b):
    return pl.pallas_call(
        functools.partial(_dense2_body, first),
        grid=(NB,),
        in_specs=[
            pl.BlockSpec((BN, D), lambda i: (i, 0)),
            pl.BlockSpec((BN, D), lambda i: (i, 0)),
            pl.BlockSpec((BN, D), lambda i: (i, 0)),
            pl.BlockSpec((8, D), lambda i: (0, 0)),
            pl.BlockSpec((1, D), lambda i: (0, 0)),
            pl.BlockSpec((1, D), lambda i: (0, 0)),
            pl.BlockSpec((D, D), lambda i: (0, 0)),
            pl.BlockSpec((1, D), lambda i: (0, 0)),
        ],
        out_specs=[pl.BlockSpec((BN, D), lambda i: (i, 0)),
                   pl.BlockSpec((8, D), lambda i: (0, 0))],
        out_shape=[jax.ShapeDtypeStruct((N, D), jnp.float32),
                   jax.ShapeDtypeStruct((8, D), jnp.float32)],
    )(agg0, agg1, hx, stats, gamma, beta, nn_W, nn_b)


def _pool_body(hx_ref, stats_ref, g_ref, b_ref, batch_ref, ow_ref, ob_ref,
               out_ref, acc_ref):
    i = pl.program_id(0)
    x = _norm_x(hx_ref[...], stats_ref[...], g_ref[...], b_ref[...])
    seg = batch_ref[0, 0]
    onehot = (seg[:, None] ==
              jax.lax.broadcasted_iota(jnp.int32, (BN, G), 1)
              ).astype(jnp.float32)
    part = jax.lax.dot_general(onehot, x, (((0,), (0,)), ((), ())),
                               preferred_element_type=jnp.float32)

    @pl.when(i == 0)
    def _():
        acc_ref[...] = part

    @pl.when(i > 0)
    def _():
        acc_ref[...] += part

    @pl.when(i == NB - 1)
    def _():
        out_ref[...] = jnp.dot(acc_ref[...], ow_ref[...],
                               preferred_element_type=jnp.float32) + ob_ref[...]


def _pool(hx, stats, gamma, beta, batch3, out_W, out_b):
    return pl.pallas_call(
        _pool_body,
        grid=(NB,),
        in_specs=[
            pl.BlockSpec((BN, D), lambda i: (i, 0)),
            pl.BlockSpec((8, D), lambda i: (0, 0)),
            pl.BlockSpec((1, D), lambda i: (0, 0)),
            pl.BlockSpec((1, D), lambda i: (0, 0)),
            pl.BlockSpec((1, 1, BN), lambda i: (i, 0, 0)),
            pl.BlockSpec((D, 6), lambda i: (0, 0)),
            pl.BlockSpec((1, 6), lambda i: (0, 0)),
        ],
        out_specs=pl.BlockSpec((G, 6), lambda i: (0, 0)),
        out_shape=jax.ShapeDtypeStruct((G, 6), jnp.float32),
        scratch_shapes=[pltpu.VMEM((G, G), jnp.float32)],
    )(hx, stats, gamma, beta, batch3, out_W, out_b)


_MESH = plsc.VectorSubcoreMesh(core_axis_name="c", subcore_axis_name="s")
_SC_PARAMS = pltpu.CompilerParams(needs_layout_passes=False)

_GDN = lax.GatherDimensionNumbers(offset_dims=(), collapsed_slice_dims=(0,),
                                  start_index_map=(0,))


def _vtake(x, idx):
    return lax.gather(x, idx[:, None], dimension_numbers=_GDN,
                      slice_sizes=(1,),
                      mode=lax.GatherScatterMode.PROMISE_IN_BOUNDS)


def _bin_body(row_hbm, col_hbm, aw_hbm, br_hbm, bc_hbm, ba_hbm, cnt_hbm,
              st_r, st_c, st_a, buf_r, buf_c, buf_a, cv_v):
    wid = lax.axis_index("c") * NS + lax.axis_index("s")
    ebase = wid * EPW
    iota = lax.broadcasted_iota(jnp.int32, (16,), 0)

    def group(g, carry):
        @pl.when(lax.rem(g, 32) == 0)
        def _():
            soff = pl.multiple_of(ebase + (g // 32) * 512, 8)
            pltpu.sync_copy(row_hbm.at[pl.ds(soff, 512)], st_r)
            pltpu.sync_copy(col_hbm.at[pl.ds(soff, 512)], st_c)
            pltpu.sync_copy(aw_hbm.at[pl.ds(soff, 512)], st_a)

        gl = lax.rem(g, 32) * 16
        rv = st_r[pl.ds(gl, 16)]
        cv = st_c[pl.ds(gl, 16)]
        av = st_a[pl.ds(gl, 16)]
        nvalid = jnp.where(g == NG - 1, EPW - (NG - 1) * 16, 16)
        valid = iota < nvalid
        bkt = cv // OWN
        out = []
        for k in range(KB):
            ck, hk = carry[2 * k], carry[2 * k + 1]
            m = (bkt == k) & valid
            plsc.store_compressed(buf_r.at[pl.ds(k * 160 + ck, 16)], rv, mask=m)
            plsc.store_compressed(buf_c.at[pl.ds(k * 160 + ck, 16)], cv, mask=m)
            plsc.store_compressed(buf_a.at[pl.ds(k * 160 + ck, 16)], av, mask=m)
            pc = plsc.all_reduce_population_count(m)
            if getattr(pc, "ndim", 0):
                pc = jnp.max(pc)
            ck = ck + pc
            do = ck >= EB
            dst = (wid * KB + k) * CAP

            @pl.when(do)
            def _(k=k, hk=hk, dst=dst):
                doff = pl.multiple_of(dst + hk, 8)
                pltpu.sync_copy(buf_r.at[pl.ds(k * 160, EB)],
                                br_hbm.at[pl.ds(doff, EB)])
                pltpu.sync_copy(buf_c.at[pl.ds(k * 160, EB)],
                                bc_hbm.at[pl.ds(doff, EB)])
                pltpu.sync_copy(buf_a.at[pl.ds(k * 160, EB)],
                                ba_hbm.at[pl.ds(doff, EB)])
                buf_r[pl.ds(k * 160, 16)] = buf_r[pl.ds(k * 160 + EB, 16)]
                buf_c[pl.ds(k * 160, 16)] = buf_c[pl.ds(k * 160 + EB, 16)]
                buf_a[pl.ds(k * 160, 16)] = buf_a[pl.ds(k * 160 + EB, 16)]

            out.append(jnp.where(do, ck - EB, ck))
            out.append(jnp.where(do, hk + EB, hk))
        return tuple(out)

    carry = lax.fori_loop(0, NG, group, (jnp.int32(0),) * (2 * KB))

    cvec = jnp.zeros((16,), jnp.int32)
    for k in range(KB):
        ck, hk = carry[2 * k], carry[2 * k + 1]
        doff = pl.multiple_of((wid * KB + k) * CAP + hk, 8)
        pltpu.sync_copy(buf_r.at[pl.ds(k * 160, EB)],
                        br_hbm.at[pl.ds(doff, EB)])
        pltpu.sync_copy(buf_c.at[pl.ds(k * 160, EB)],
                        bc_hbm.at[pl.ds(doff, EB)])
        pltpu.sync_copy(buf_a.at[pl.ds(k * 160, EB)],
                        ba_hbm.at[pl.ds(doff, EB)])
        cvec = jnp.where(iota == k, ck + hk, cvec)
    cv_v[...] = cvec
    pltpu.sync_copy(cv_v, cnt_hbm.at[pl.ds(pl.multiple_of(wid * 16, 8), 16)])


_bin = pl.kernel(
    _bin_body,
    out_type=[jax.ShapeDtypeStruct((NW * KB * CAP,), jnp.int32),
              jax.ShapeDtypeStruct((NW * KB * CAP,), jnp.int32),
              jax.ShapeDtypeStruct((NW * KB * CAP,), jnp.float32),
              jax.ShapeDtypeStruct((NW * 16,), jnp.int32)],
    mesh=_MESH,
    compiler_params=_SC_PARAMS,
    scratch_types=[pltpu.VMEM((512,), jnp.int32),
                   pltpu.VMEM((512,), jnp.int32),
                   pltpu.VMEM((512,), jnp.float32),
                   pltpu.VMEM((KB * 160,), jnp.int32),
                   pltpu.VMEM((KB * 160,), jnp.int32),
                   pltpu.VMEM((KB * 160,), jnp.float32),
                   pltpu.VMEM((16,), jnp.int32)],
)


def _edge_body(u_hbm, v_hbm, vl_hbm, br_hbm, bc_hbm, ba_hbm, cnt_hbm,
               zin_hbm, agg_hbm,
               st_r, st_c, st_a, rbuf, cbuf, abuf, lbuf,
               ubuf, vbuf, vlv, cntv, acc, sem_u, sem_v):
    c_ax = lax.axis_index("c")
    s_ax = lax.axis_index("s")
    iota = lax.broadcasted_iota(jnp.int32, (16,), 0)
    pltpu.sync_copy(vl_hbm, vlv)
    pltpu.sync_copy(cnt_hbm, cntv)
    vl_ts = [vlv[pl.ds(t * 16, 16)] for t in range(16)]
    colv_ts = [iota + t * 16 for t in range(16)]
    tile_base = s_ax * OWN
    ns = []
    for w in range(NS):
        cvec = cntv[pl.ds(pl.multiple_of((c_ax * NS + w) * 16, 8), 16)]
        ns.append(jnp.sum(jnp.where(iota == s_ax, cvec, 0)))

    def fire(ck, sub_base, tail):
        for t in range(EB // 16):
            rv = jnp.clip(rbuf[pl.ds(t * 16, 16)], 0, N - 1)
            rbuf[pl.ds(t * 16, 16)] = rv
            cv = jnp.clip(cbuf[pl.ds(t * 16, 16)], 0, N - 1)
            cbuf[pl.ds(t * 16, 16)] = cv
            lv = cv - sub_base
            if tail:
                lv = jnp.where(iota + t * 16 < ck, lv, CT)
            lbuf[pl.ds(t * 16, 16)] = jnp.clip(lv, 0, CT)
        cpu_ = pltpu.async_copy(u_hbm.at[rbuf.at[pl.ds(0, EB)]], ubuf, sem_u)
        cpv_ = pltpu.async_copy(v_hbm.at[cbuf.at[pl.ds(0, EB)]], vbuf, sem_v)
        cpu_.wait()
        cpv_.wait()

        def edge_j(jv, _):
            jj = jv // 16
            jl = jv - jj * 16
            splat = jnp.zeros((16,), jnp.int32) + jl
            av = abuf[pl.ds(pl.multiple_of(jj * 16, 8), 16)]
            ajv = _vtake(av, splat)
            lv = lbuf[pl.ds(pl.multiple_of(jj * 16, 8), 16)]
            rowv = _vtake(lv, splat)
            for t in range(16):
                m = ubuf[jv, pl.ds(t * 16, 16)] + vbuf[jv, pl.ds(t * 16, 16)]
                m = jnp.maximum(m + ajv * vl_ts[t], 0.0)
                plsc.addupdate_scatter(acc, [rowv, colv_ts[t]], m)
            return 0

        lax.fori_loop(0, EB, edge_j, 0)

    def sub_body(sub, _):
        pltpu.sync_copy(zin_hbm, acc)
        sub_lo = sub * CT

        def w_body(w, ck):
            cvec = cntv[pl.ds(pl.multiple_of((c_ax * NS + w) * 16, 8), 16)]
            n_w = jnp.sum(jnp.where(iota == s_ax, cvec, 0))
            loff = ((c_ax * NS + w) * KB + s_ax) * CAP

            def stage(st, ck):
                soff = pl.multiple_of(loff + st * 512, 8)
                pltpu.sync_copy(br_hbm.at[pl.ds(soff, 512)], st_r)
                pltpu.sync_copy(bc_hbm.at[pl.ds(soff, 512)], st_c)
                pltpu.sync_copy(ba_hbm.at[pl.ds(soff, 512)], st_a)

                def group(g, ck):
                    gl = g * 16
                    rv = st_r[pl.ds(gl, 16)]
                    cv = st_c[pl.ds(gl, 16)]
                    av = st_a[pl.ds(gl, 16)]
                    glane = st * 512 + gl + iota
                    lrow = cv - tile_base - sub_lo
                    m = (glane < n_w) & (lrow >= 0) & (lrow < CT)
                    plsc.store_compressed(rbuf.at[pl.ds(ck, 16)], rv, mask=m)
                    plsc.store_compressed(cbuf.at[pl.ds(ck, 16)], cv, mask=m)
                    plsc.store_compressed(abuf.at[pl.ds(ck, 16)], av, mask=m)
                    pc = plsc.all_reduce_population_count(m)
                    if getattr(pc, "ndim", 0):
                        pc = jnp.max(pc)
                    ck = ck + pc
                    do = ck >= EB

                    @pl.when(do)
                    def _():
                        fire(EB, tile_base + sub_lo, False)
                        rbuf[pl.ds(0, 16)] = rbuf[pl.ds(EB, 16)]
                        cbuf[pl.ds(0, 16)] = cbuf[pl.ds(EB, 16)]
                        abuf[pl.ds(0, 16)] = abuf[pl.ds(EB, 16)]

                    return jnp.where(do, ck - EB, ck)

                return lax.fori_loop(0, 32, group, ck)

            return lax.fori_loop(0, (n_w + 511) // 512, stage, ck)

        ck = lax.fori_loop(0, NS, w_body, jnp.int32(0))

        @pl.when(ck > 0)
        def _():
            fire(ck, tile_base + sub_lo, True)

        dst = pl.multiple_of(c_ax * AGGP + tile_base + sub_lo, 8)
        pltpu.sync_copy(acc.at[pl.ds(0, CT)], agg_hbm.at[pl.ds(dst, CT)])
        return 0

    lax.fori_loop(0, SUBS, sub_body, 0)


_edge_sc = pl.kernel(
    _edge_body,
    out_type=jax.ShapeDtypeStruct((NC * AGGP, D), jnp.float32),
    mesh=_MESH,
    compiler_params=_SC_PARAMS,
    scratch_types=[pltpu.VMEM((512,), jnp.int32),
                   pltpu.VMEM((512,), jnp.int32),
                   pltpu.VMEM((512,), jnp.float32),
                   pltpu.VMEM((EB + 32,), jnp.int32),
                   pltpu.VMEM((EB + 32,), jnp.int32),
                   pltpu.VMEM((EB + 32,), jnp.float32),
                   pltpu.VMEM((EB,), jnp.int32),
                   pltpu.VMEM((EB, D), jnp.float32),
                   pltpu.VMEM((EB, D), jnp.float32),
                   pltpu.VMEM((D,), jnp.float32),
                   pltpu.VMEM((NW * 16,), jnp.int32),
                   pltpu.VMEM((CTA, D), jnp.float32),
                   pltpu.SemaphoreType.DMA,
                   pltpu.SemaphoreType.DMA],
)


def kernel(node_features, edge_index, atom_weights, batch, atom_W, atom_b,
           edge_W, edge_b, mlp_W, mlp_b, nn_W, nn_b, bn_gamma, bn_beta,
           out_W, out_b):
    f32 = jnp.float32
    W1, W2, W3 = mlp_W[:D], mlp_W[D:2 * D], mlp_W[2 * D:]
    row = edge_index[0]
    col = edge_index[1]
    aw = atom_weights[:, 0]

    nf_pad = jnp.pad(node_features, ((0, 0), (0, 128 - node_features.shape[1])))
    atom_Wp = jnp.pad(atom_W, ((0, 128 - atom_W.shape[0]), (0, 0)))
    gamma = bn_gamma.reshape(1, D)
    beta = bn_beta.reshape(1, D)
    batch3 = batch.astype(jnp.int32).reshape(NB, 1, BN)

    rowp = jnp.pad(row.astype(jnp.int32), (0, EPAD - E))
    colp = jnp.pad(col.astype(jnp.int32), (0, EPAD - E))
    awp = jnp.pad(aw, (0, EPAD - E))
    zin = jnp.zeros((CTA, D), f32)
    br, bc, ba, cnts = _bin(rowp, colp, awp)

    vs, cs = _vc(edge_W, edge_b, mlp_b, W3)

    hx = _x0(nf_pad, atom_Wp, atom_b.reshape(1, D))
    stats = jnp.zeros((8, D), f32)
    P = Q = jnp.zeros((N, D), f32)
    for l in range(LAYERS):
        first = (l == 0)
        P, Q, U, V = _prep(first, hx, stats, gamma, beta, P, Q,
                           W1, W2, W3, cs[l].reshape(1, D))
        agg2 = _edge_sc(U, V, vs[l], br, bc, ba, cnts, zin)
        hx, stats = _dense2(first, agg2[:N], agg2[AGGP:AGGP + N],
                            hx, stats, gamma, beta,
                            nn_W, nn_b.reshape(1, D))
    return _pool(hx, stats, gamma, beta, batch3, out_W, out_b.reshape(1, 6))
